# baseline (speedup 1.0000x reference)
import functools

import jax
import jax.numpy as jnp
from jax.experimental import pallas as pl

_TILE_N = 2000


def _logits_body(w_ref, xt_ref, b_ref, o_ref):
    acc = jax.lax.dot_general(
        w_ref[...], xt_ref[...],
        dimension_numbers=(((1,), (0,)), ((), ())),
        preferred_element_type=jnp.float32,
    )
    o_ref[...] = acc + b_ref[...]


@functools.partial(jax.jit, static_argnames=())
def kernel(inputs, labels, W, b):
    del labels
    B, D = inputs.shape
    N = W.shape[0]
    grid = N // _TILE_N
    w16 = W.astype(jnp.bfloat16)
    xt = inputs.T.astype(jnp.bfloat16)
    b2 = b.reshape(N, 1)
    outT = pl.pallas_call(
        _logits_body,
        grid=(grid,),
        in_specs=[
            pl.BlockSpec((_TILE_N, D), lambda i: (i, 0)),
            pl.BlockSpec((D, B), lambda i: (0, 0)),
            pl.BlockSpec((_TILE_N, 1), lambda i: (i, 0)),
        ],
        out_specs=pl.BlockSpec((_TILE_N, B), lambda i: (i, 0)),
        out_shape=jax.ShapeDtypeStruct((N, B), jnp.float32),
    )(w16, xt, b2)
    return outT.T


# speedup vs baseline: 1.0022x; 1.0022x over previous
import functools

import jax
import jax.numpy as jnp
from jax.experimental import pallas as pl
from jax.experimental.pallas import tpu as pltpu

_TILE_N = 2000
_NBUF = 4


def _logits_body(w_ref, xt_ref, b_ref, o_hbm, acc_ref, sem):
    i = pl.program_id(0)
    nsteps = pl.num_programs(0)
    slot = jax.lax.rem(i, _NBUF)

    @pl.when(i >= _NBUF)
    def _():
        pltpu.make_async_copy(
            acc_ref.at[slot], o_hbm.at[pl.ds(0, _TILE_N), :], sem.at[slot]
        ).wait()

    acc = jax.lax.dot_general(
        w_ref[...], xt_ref[...],
        dimension_numbers=(((1,), (0,)), ((), ())),
        preferred_element_type=jnp.float32,
    )
    acc_ref[slot] = acc + b_ref[...]

    pltpu.make_async_copy(
        acc_ref.at[slot], o_hbm.at[pl.ds(i * _TILE_N, _TILE_N), :], sem.at[slot]
    ).start()

    @pl.when(i == nsteps - 1)
    def _():
        for s in range(_NBUF):
            @pl.when(jnp.asarray(s) < jnp.minimum(nsteps, _NBUF))
            def _(s=s):
                pltpu.make_async_copy(
                    acc_ref.at[s], o_hbm.at[pl.ds(0, _TILE_N), :], sem.at[s]
                ).wait()


@functools.partial(jax.jit, static_argnames=())
def kernel(inputs, labels, W, b):
    del labels
    B, D = inputs.shape
    N = W.shape[0]
    grid = N // _TILE_N
    w16 = W.astype(jnp.bfloat16)
    xt = inputs.T.astype(jnp.bfloat16)
    b2 = b.reshape(N, 1)
    outT = pl.pallas_call(
        _logits_body,
        grid=(grid,),
        in_specs=[
            pl.BlockSpec((_TILE_N, D), lambda i: (i, 0)),
            pl.BlockSpec((D, B), lambda i: (0, 0)),
            pl.BlockSpec((_TILE_N, 1), lambda i: (i, 0)),
        ],
        out_specs=pl.BlockSpec(memory_space=pl.ANY),
        out_shape=jax.ShapeDtypeStruct((N, B), jnp.float32),
        scratch_shapes=[
            pltpu.VMEM((_NBUF, _TILE_N, B), jnp.float32),
            pltpu.SemaphoreType.DMA((_NBUF,)),
        ],
    )(w16, xt, b2)
    return outT.T
